# baseline (speedup 1.0000x reference)
import functools

import jax
import jax.numpy as jnp
from jax.experimental import pallas as pl
from jax.experimental.pallas import tpu as pltpu

TOPK = 16
NEG_INF = float("-inf")


def _topk_body(lite_ref, idx_ref):
    scores = jnp.max(lite_ref[...], axis=2)
    t_iota = jax.lax.broadcasted_iota(jnp.int32, scores.shape, 1)
    k_iota = jax.lax.broadcasted_iota(jnp.int32, (1, TOPK), 1)
    ind_row = jnp.zeros((1, TOPK), jnp.int32)
    for k in range(TOPK):
        idx = jnp.argmax(scores, axis=1)[0].astype(jnp.int32)
        ind_row = jnp.where(k_iota == k, idx, ind_row)
        scores = jnp.where(t_iota == idx, NEG_INF, scores)
    idx_ref[...] = ind_row.reshape(1, 1, TOPK)


def _gather_mean_body(idx_ref, in_ref, out_ref):
    k = pl.program_id(1)

    @pl.when(k == 0)
    def _():
        out_ref[...] = jnp.zeros_like(out_ref)

    out_ref[...] += in_ref[...]

    @pl.when(k == TOPK - 1)
    def _():
        out_ref[...] *= 1.0 / TOPK


@jax.jit
def kernel(input, lite_input):
    B, T, D = input.shape

    indices = pl.pallas_call(
        _topk_body,
        grid=(B,),
        in_specs=[pl.BlockSpec((1, T, D), lambda b: (b, 0, 0))],
        out_specs=pl.BlockSpec((1, 1, TOPK), lambda b: (b, 0, 0)),
        out_shape=jax.ShapeDtypeStruct((B, 1, TOPK), jnp.int32),
    )(lite_input)

    input_rows = input.reshape(B * T, 1, D)
    out = pl.pallas_call(
        _gather_mean_body,
        grid_spec=pltpu.PrefetchScalarGridSpec(
            num_scalar_prefetch=1,
            grid=(B, TOPK),
            in_specs=[
                pl.BlockSpec(
                    (1, 1, D), lambda b, k, idx: (b * T + idx[b, 0, k], 0, 0)
                ),
            ],
            out_specs=pl.BlockSpec((1, 1, D), lambda b, k, idx: (b, 0, 0)),
        ),
        out_shape=jax.ShapeDtypeStruct((B, 1, D), jnp.float32),
    )(indices, input_rows)

    return out


# speedup vs baseline: 15.1092x; 15.1092x over previous
import jax
import jax.numpy as jnp
from jax.experimental import pallas as pl

TOPK = 16
BB = 8
NEG_INF = float("-inf")


def _consensus_body(lite_ref, in_ref, out_ref):
    scores = jnp.max(lite_ref[...], axis=2)
    t_iota = jax.lax.broadcasted_iota(jnp.int32, scores.shape, 1)
    selected = jnp.zeros(scores.shape, jnp.bool_)
    big = jnp.int32(2**30)
    for _ in range(TOPK):
        m = jnp.max(scores, axis=1, keepdims=True)
        cand = jnp.where(scores == m, t_iota, big)
        idx = jnp.min(cand, axis=1, keepdims=True)
        hit = t_iota == idx
        selected = jnp.logical_or(selected, hit)
        scores = jnp.where(hit, NEG_INF, scores)
    w = jnp.where(selected, 1.0 / TOPK, 0.0)
    acc = jnp.sum(in_ref[...] * w[:, :, None], axis=1, keepdims=True)
    out_ref[...] = acc


@jax.jit
def kernel(input, lite_input):
    B, T, D = input.shape

    out = pl.pallas_call(
        _consensus_body,
        grid=(B // BB,),
        in_specs=[
            pl.BlockSpec((BB, T, D), lambda b: (b, 0, 0)),
            pl.BlockSpec((BB, T, D), lambda b: (b, 0, 0)),
        ],
        out_specs=pl.BlockSpec((BB, 1, D), lambda b: (b, 0, 0)),
        out_shape=jax.ShapeDtypeStruct((B, 1, D), jnp.float32),
    )(lite_input, input)

    return out
